# baseline (speedup 1.0000x reference)
import functools

import jax
import jax.numpy as jnp
from jax import lax
from jax.experimental import pallas as pl
from jax.experimental.pallas import tpu as pltpu
from jax.experimental.pallas import tpu_sc as plsc

B = 16384
F = 26
V = 100000
D = 16

NC = 2
NS = 16
NW = NC * NS
NPW = B // NW
CH = 128
NCH = NPW // CH
NG = CH // 16


def _fm_body(idx_hbm, lin_hbm, fm_hbm, out_hbm,
             idx_v, lin_v, rows_v, tbuf, out_v, sem_f, sem_l):
    wid = lax.axis_index("s") * NC + lax.axis_index("c")
    wbase = wid * NPW

    @pl.loop(0, NCH)
    def _chunk(ci):
        base = wbase + ci * CH
        pltpu.sync_copy(idx_hbm.at[:, pl.ds(base, CH)], idx_v)

        @pl.loop(0, F)
        def _fire(f):
            pltpu.async_copy(fm_hbm.at[idx_v.at[f]], rows_v.at[f], sem_f)
            pltpu.async_copy(lin_hbm.at[idx_v.at[f]], lin_v.at[f], sem_l)

        @pl.loop(0, F)
        def _drain(f):
            pltpu.make_async_copy(fm_hbm.at[idx_v.at[f]], rows_v.at[f], sem_f).wait()
            pltpu.make_async_copy(lin_hbm.at[idx_v.at[f]], lin_v.at[f], sem_l).wait()

        @pl.loop(0, NG)
        def _grp(g):
            gbase = g * 16
            for jj in range(16):
                j = gbase + jj
                v = rows_v[0, j, :]
                s = v
                q = v * v
                for f in range(1, F):
                    v = rows_v[f, j, :]
                    s = s + v
                    q = q + v * v
                tbuf[pl.ds(jj * 16, 16)] = 0.5 * (s * s - q)
            res = lin_v[0, pl.ds(gbase, 16)]
            for f in range(1, F):
                res = res + lin_v[f, pl.ds(gbase, 16)]
            rows16 = lax.iota(jnp.int32, 16) * 16
            for d in range(D):
                col = plsc.load_gather(tbuf, [rows16 + d])
                res = res + col
            out_v[pl.ds(gbase, 16)] = res

        pltpu.sync_copy(out_v, out_hbm.at[pl.ds(base, CH)])


@functools.partial(jax.jit, static_argnames=("interpret",))
def _fm_call(idx_t, lin_flat, fm_flat, interpret=False):
    mesh = plsc.VectorSubcoreMesh(core_axis_name="c", subcore_axis_name="s",
                                  num_cores=NC, num_subcores=NS)
    return pl.kernel(
        _fm_body,
        out_type=jax.ShapeDtypeStruct((B,), jnp.float32),
        mesh=mesh,
        scratch_types=[
            pltpu.VMEM((F, CH), jnp.int32),
            pltpu.VMEM((F, CH), jnp.float32),
            pltpu.VMEM((F, CH, D), jnp.float32),
            pltpu.VMEM((16 * 16,), jnp.float32),
            pltpu.VMEM((CH,), jnp.float32),
            pltpu.SemaphoreType.DMA,
            pltpu.SemaphoreType.DMA,
        ],
        compiler_params=pltpu.CompilerParams(needs_layout_passes=False,
                                             use_tc_tiling_on_sc=False),
        interpret=interpret,
    )(idx_t, lin_flat, fm_flat)


def kernel(X, lin_W, fm_W):
    idx_t = (X.astype(jnp.int32)
             + (jnp.arange(F, dtype=jnp.int32) * V)[None, :]).T
    lin_flat = lin_W.reshape(F * V)
    fm_flat = fm_W.reshape(F * V, D)
    return _fm_call(idx_t, lin_flat, fm_flat)


# speedup vs baseline: 1.0148x; 1.0148x over previous
import functools

import jax
import jax.numpy as jnp
from jax import lax
from jax.experimental import pallas as pl
from jax.experimental.pallas import tpu as pltpu
from jax.experimental.pallas import tpu_sc as plsc

B = 16384
F = 26
V = 100000
D = 16

NC = 2
NS = 16
NW = NC * NS
NPW = B // NW
CH = 128
NCH = NPW // CH
NG = CH // 16


def _fm_body(idx_hbm, lin_hbm, fm_hbm, out_hbm,
             idx_v, lin_v, rows_v, tbuf, out_v,
             sem_f0, sem_l0, sem_f1, sem_l1):
    wid = lax.axis_index("s") * NC + lax.axis_index("c")
    wbase = wid * NPW
    gc0 = wid * NCH
    sems = ((sem_f0, sem_l0), (sem_f1, sem_l1))

    def fire(ci, slot):
        sem_f, sem_l = sems[slot]
        pltpu.sync_copy(idx_hbm.at[gc0 + ci], idx_v.at[slot])

        @pl.loop(0, F)
        def _fire(f):
            pltpu.async_copy(fm_hbm.at[idx_v.at[slot, f]],
                             rows_v.at[slot, f], sem_f)
            pltpu.async_copy(lin_hbm.at[idx_v.at[slot, f]],
                             lin_v.at[slot, f], sem_l)

    def drain(ci, slot):
        sem_f, sem_l = sems[slot]

        @pl.loop(0, F)
        def _drain(f):
            pltpu.make_async_copy(fm_hbm.at[idx_v.at[slot, f]],
                                  rows_v.at[slot, f], sem_f).wait()
            pltpu.make_async_copy(lin_hbm.at[idx_v.at[slot, f]],
                                  lin_v.at[slot, f], sem_l).wait()

    def compute(ci, slot):
        @pl.loop(0, NG)
        def _grp(g):
            gbase = g * 16
            for jj in range(16):
                j = gbase + jj
                acc_s = [None] * 4
                acc_q = [None] * 4
                for f in range(F):
                    v = rows_v[slot, f, j, :]
                    k = f % 4
                    if acc_s[k] is None:
                        acc_s[k] = v
                        acc_q[k] = v * v
                    else:
                        acc_s[k] = acc_s[k] + v
                        acc_q[k] = acc_q[k] + v * v
                s = (acc_s[0] + acc_s[1]) + (acc_s[2] + acc_s[3])
                q = (acc_q[0] + acc_q[1]) + (acc_q[2] + acc_q[3])
                tbuf[pl.ds(jj * 16, 16)] = 0.5 * (s * s - q)
            res = lin_v[slot, 0, pl.ds(gbase, 16)]
            for f in range(1, F):
                res = res + lin_v[slot, f, pl.ds(gbase, 16)]
            rows16 = lax.iota(jnp.int32, 16) * 16
            for d in range(D):
                res = res + plsc.load_gather(tbuf, [rows16 + d])
            out_v[pl.ds(gbase, 16)] = res

        pltpu.sync_copy(out_v, out_hbm.at[pl.ds(wbase + ci * CH, CH)])

    fire(0, 0)
    for c in range(NCH):
        if c + 1 < NCH:
            fire(c + 1, (c + 1) % 2)
        drain(c, c % 2)
        compute(c, c % 2)


@functools.partial(jax.jit, static_argnames=("interpret",))
def _fm_call(idx3, lin_flat, fm_flat, interpret=False):
    mesh = plsc.VectorSubcoreMesh(core_axis_name="c", subcore_axis_name="s",
                                  num_cores=NC, num_subcores=NS)
    return pl.kernel(
        _fm_body,
        out_type=jax.ShapeDtypeStruct((B,), jnp.float32),
        mesh=mesh,
        scratch_types=[
            pltpu.VMEM((2, F, CH), jnp.int32),
            pltpu.VMEM((2, F, CH), jnp.float32),
            pltpu.VMEM((2, F, CH, D), jnp.float32),
            pltpu.VMEM((16 * 16,), jnp.float32),
            pltpu.VMEM((CH,), jnp.float32),
            pltpu.SemaphoreType.DMA,
            pltpu.SemaphoreType.DMA,
            pltpu.SemaphoreType.DMA,
            pltpu.SemaphoreType.DMA,
        ],
        compiler_params=pltpu.CompilerParams(needs_layout_passes=False,
                                             use_tc_tiling_on_sc=False),
        interpret=interpret,
    )(idx3, lin_flat, fm_flat)


def kernel(X, lin_W, fm_W):
    idx_t = (X.astype(jnp.int32)
             + (jnp.arange(F, dtype=jnp.int32) * V)[None, :]).T
    idx3 = idx_t.reshape(F, B // CH, CH).transpose(1, 0, 2)
    lin_flat = lin_W.reshape(F * V)
    fm_flat = fm_W.reshape(F * V, D)
    return _fm_call(idx3, lin_flat, fm_flat)


# speedup vs baseline: 1.7093x; 1.6843x over previous
import functools

import jax
import jax.numpy as jnp
from jax import lax
from jax.experimental import pallas as pl
from jax.experimental.pallas import tpu as pltpu
from jax.experimental.pallas import tpu_sc as plsc

B = 16384
F = 26
V = 100000
D = 16

NC = 2
NS = 16
NW = NC * NS
NPW = B // NW
CH = 128
NCH = NPW // CH
NG = CH // 16


def _fm_body(idx_hbm, lin_hbm, fm_hbm, out_hbm,
             idx_v, lin_v, fm_v, out_v,
             sem_f0, sem_l0, sem_f1, sem_l1):
    wid = lax.axis_index("s") * NC + lax.axis_index("c")
    wbase = wid * NPW
    gc0 = wid * NCH
    sems = ((sem_f0, sem_l0), (sem_f1, sem_l1))

    def fire(ci, slot):
        sem_f, sem_l = sems[slot]
        pltpu.sync_copy(idx_hbm.at[gc0 + ci], idx_v.at[slot])

        @pl.loop(0, F)
        def _fire_f(f):
            pltpu.async_copy(lin_hbm.at[f].at[idx_v.at[slot, f]],
                             lin_v.at[slot, f], sem_l)

            @pl.loop(0, D)
            def _fire_d(d):
                pltpu.async_copy(fm_hbm.at[f * D + d].at[idx_v.at[slot, f]],
                                 fm_v.at[slot, f * D + d], sem_f)

    def drain(ci, slot):
        sem_f, sem_l = sems[slot]

        @pl.loop(0, F)
        def _drain_f(f):
            pltpu.make_async_copy(lin_hbm.at[f].at[idx_v.at[slot, f]],
                                  lin_v.at[slot, f], sem_l).wait()

            @pl.loop(0, D)
            def _drain_d(d):
                pltpu.make_async_copy(fm_hbm.at[f * D + d].at[idx_v.at[slot, f]],
                                      fm_v.at[slot, f * D + d], sem_f).wait()

    def compute(ci, slot):
        @pl.loop(0, NG)
        def _grp(g):
            sl = pl.ds(g * 16, 16)
            res = lin_v[slot, 0, sl]
            for f in range(1, F):
                res = res + lin_v[slot, f, sl]
            for d in range(D):
                acc_s = [None] * 4
                acc_q = [None] * 4
                for f in range(F):
                    v = fm_v[slot, f * D + d, sl]
                    k = f % 4
                    if acc_s[k] is None:
                        acc_s[k] = v
                        acc_q[k] = v * v
                    else:
                        acc_s[k] = acc_s[k] + v
                        acc_q[k] = acc_q[k] + v * v
                s = (acc_s[0] + acc_s[1]) + (acc_s[2] + acc_s[3])
                q = (acc_q[0] + acc_q[1]) + (acc_q[2] + acc_q[3])
                res = res + 0.5 * (s * s - q)
            out_v[sl] = res

        pltpu.sync_copy(out_v, out_hbm.at[pl.ds(wbase + ci * CH, CH)])

    fire(0, 0)
    for c in range(NCH):
        if c + 1 < NCH:
            fire(c + 1, (c + 1) % 2)
        drain(c, c % 2)
        compute(c, c % 2)


@functools.partial(jax.jit, static_argnames=("interpret",))
def _fm_call(idx3, lin2d, fm2d, interpret=False):
    mesh = plsc.VectorSubcoreMesh(core_axis_name="c", subcore_axis_name="s",
                                  num_cores=NC, num_subcores=NS)
    return pl.kernel(
        _fm_body,
        out_type=jax.ShapeDtypeStruct((B,), jnp.float32),
        mesh=mesh,
        scratch_types=[
            pltpu.VMEM((2, F, CH), jnp.int32),
            pltpu.VMEM((2, F, CH), jnp.float32),
            pltpu.VMEM((2, F * D, CH), jnp.float32),
            pltpu.VMEM((CH,), jnp.float32),
            pltpu.SemaphoreType.DMA,
            pltpu.SemaphoreType.DMA,
            pltpu.SemaphoreType.DMA,
            pltpu.SemaphoreType.DMA,
        ],
        compiler_params=pltpu.CompilerParams(needs_layout_passes=False,
                                             use_tc_tiling_on_sc=False),
        interpret=interpret,
    )(idx3, lin2d, fm2d)


def kernel(X, lin_W, fm_W):
    idx3 = X.T.reshape(F, B // CH, CH).transpose(1, 0, 2)
    lin2d = lin_W.reshape(F, V)
    fm2d = fm_W.transpose(0, 2, 1).reshape(F * D, V)
    return _fm_call(idx3, lin2d, fm2d)
